# baseline (speedup 1.0000x reference)
import functools

import jax
import jax.numpy as jnp
from jax import lax
from jax.experimental import pallas as pl
from jax.experimental.pallas import tpu as pltpu
from jax.experimental.pallas import tpu_sc as plsc

N = 10000
E = 160000
D = 256

NC = 2
NS = 16
DH = D // NC
CHUNK = 128
GSZ = 2
GSZD = 8
NCHUNK_T = 80
NCHUNKP = NS * NCHUNK_T
EPAD = NCHUNKP * CHUNK
NIT = NCHUNK_T // (2 * GSZ)
NIT_DEG = (NCHUNK_T // 2 - GSZD) // (2 * GSZD)
PAD = 48
NPR = N + PAD
ZB = 8
WB = 40


def _sc_agg_body(x2_hbm, ei_hbm, zrow_hbm,
                 agg0_out, agg1_out,
                 src4_a, dst4_a, src4_b, dst4_b, gidx4,
                 r0, r1, agg_sh,
                 isem_a, isem_b, gsem0, gsem1,
                 ssem0, ssem1):
    cid = lax.axis_index("c")
    sid = lax.axis_index("s")
    rows = (r0, r1)
    gsems = (gsem0, gsem1)
    ssems = (ssem0, ssem1)
    base = sid * NCHUNK_T

    @pl.loop(0, (NPR // ZB + NS - 1) // NS)
    def _(it):
        z = it * NS + sid

        @pl.when(z < NPR // ZB)
        def _():
            pltpu.sync_copy(zrow_hbm, agg_sh.at[pl.ds(z * ZB, ZB)])

    plsc.subcore_barrier()

    def fetch_idx(off, src4, dst4, isem):
        pltpu.async_copy(ei_hbm.at[pl.ds(off, GSZ)], src4, isem)
        pltpu.async_copy(ei_hbm.at[pl.ds(NCHUNKP + off, GSZ)], dst4, isem)

    def wait_idx(src4, dst4, isem):
        pltpu.make_async_copy(ei_hbm.at[pl.ds(0, GSZ)], src4, isem).wait()
        pltpu.make_async_copy(ei_hbm.at[pl.ds(0, GSZ)], dst4, isem).wait()

    def compute_gidx(src4):
        for j in range(GSZ):
            @pl.loop(0, CHUNK, step=16)
            def _(i):
                s = src4[j, pl.ds(i, 16)]
                gidx4[j, pl.ds(i, 16)] = s * 2 + cid

    def half_group(dst4, prev_dst4, first):
        for j in range(GSZ):
            if first is None:
                pltpu.make_async_copy(
                    rows[j], agg_sh.at[prev_dst4.at[j]], ssems[j]).wait()
            else:
                @pl.when(first > 0)
                def _():
                    pltpu.make_async_copy(
                        rows[j], agg_sh.at[prev_dst4.at[j]],
                        ssems[j]).wait()

            pltpu.async_copy(x2_hbm.at[gidx4.at[j]], rows[j], gsems[j])

    def scatters(dst4):
        for j in range(GSZ):
            pltpu.make_async_copy(x2_hbm.at[gidx4.at[j]], rows[j],
                                  gsems[j]).wait()
            pltpu.async_copy(rows[j], agg_sh.at[dst4.at[j]], ssems[j],
                             add=True)

    fetch_idx(base, src4_a, dst4_a, isem_a)

    @pl.loop(0, NIT)
    def _(t):
        off_a = base + t * (2 * GSZ)
        wait_idx(src4_a, dst4_a, isem_a)
        compute_gidx(src4_a)
        half_group(dst4_a, dst4_b, t)
        fetch_idx(off_a + GSZ, src4_b, dst4_b, isem_b)
        scatters(dst4_a)
        wait_idx(src4_b, dst4_b, isem_b)
        compute_gidx(src4_b)
        half_group(dst4_b, dst4_a, None)

        @pl.when(t + 1 < NIT)
        def _():
            fetch_idx(off_a + 2 * GSZ, src4_a, dst4_a, isem_a)

        scatters(dst4_b)

    for j in range(GSZ):
        pltpu.make_async_copy(rows[j], agg_sh.at[dst4_b.at[j]],
                              ssems[j]).wait()

    plsc.subcore_barrier()

    @pl.loop(0, (N // WB + NS - 1) // NS)
    def _(it):
        r = it * NS + sid

        @pl.when(r < N // WB)
        def _():
            row = r * WB

            @pl.when(cid == 0)
            def _():
                pltpu.sync_copy(agg_sh.at[pl.ds(row, WB)],
                                agg0_out.at[pl.ds(row, WB)])

            @pl.when(cid == 1)
            def _():
                pltpu.sync_copy(agg_sh.at[pl.ds(row, WB)],
                                agg1_out.at[pl.ds(row, WB)])


def _sc_deg_body(ei_hbm, zrow_hbm, ones_hbm, deg_out,
                 dst4_a, dst4_b, ones_v, deg_sh,
                 isem_a, isem_b, ssem0, ssem1):
    cid = lax.axis_index("c")
    sid = lax.axis_index("s")
    ssems = tuple([ssem0] * (GSZD // 2) + [ssem1] * (GSZD - GSZD // 2))
    base = NCHUNKP + cid * (NCHUNKP // 2) + sid * (NCHUNK_T // 2)
    GS = GSZD

    @pl.loop(0, (NPR // ZB + NS - 1) // NS)
    def _(it):
        z = it * NS + sid

        @pl.when(z < NPR // ZB)
        def _():
            pltpu.sync_copy(zrow_hbm, deg_sh.at[pl.ds(z * ZB, ZB)])

    pltpu.sync_copy(ones_hbm, ones_v)
    plsc.subcore_barrier()

    def wait_idx(dst4, isem):
        pltpu.make_async_copy(ei_hbm.at[pl.ds(0, GS)], dst4, isem).wait()

    def scatter_group(dst4, prev_dst4, first):
        for j in range(GS):
            if first is None:
                pltpu.make_async_copy(
                    ones_v, deg_sh.at[prev_dst4.at[j]], ssems[j]).wait()
            else:
                @pl.when(first > 0)
                def _():
                    pltpu.make_async_copy(
                        ones_v, deg_sh.at[prev_dst4.at[j]], ssems[j]).wait()

        for j in range(GS):
            pltpu.async_copy(ones_v, deg_sh.at[dst4.at[j]], ssems[j],
                             add=True)

    pltpu.async_copy(ei_hbm.at[pl.ds(base, GS)], dst4_a, isem_a)

    @pl.loop(0, NIT_DEG)
    def _(t):
        off_a = base + t * (2 * GS)
        wait_idx(dst4_a, isem_a)
        scatter_group(dst4_a, dst4_b, t)
        pltpu.async_copy(ei_hbm.at[pl.ds(off_a + GS, GS)], dst4_b, isem_b)
        wait_idx(dst4_b, isem_b)
        scatter_group(dst4_b, dst4_a, None)
        pltpu.async_copy(ei_hbm.at[pl.ds(off_a + 2 * GS, GS)],
                         dst4_a, isem_a)

    wait_idx(dst4_a, isem_a)
    scatter_group(dst4_a, dst4_b, None)

    for j in range(GS):
        pltpu.make_async_copy(ones_v, deg_sh.at[dst4_a.at[j]],
                              ssems[j]).wait()

    plsc.subcore_barrier()

    @pl.loop(0, (N // WB + NS - 1) // NS)
    def _(it):
        r = it * NS + sid

        @pl.when(r < N // WB)
        def _():
            row = r * WB
            pltpu.sync_copy(deg_sh.at[pl.ds(row, WB)],
                            deg_out.at[cid, pl.ds(row, WB)])


@functools.cache
def _sc_fns():
    mesh = plsc.VectorSubcoreMesh(core_axis_name="c", subcore_axis_name="s")
    agg_fn = pl.kernel(
        _sc_agg_body,
        out_type=[
            jax.ShapeDtypeStruct((N, DH), jnp.float32),
            jax.ShapeDtypeStruct((N, DH), jnp.float32),
        ],
        mesh=mesh,
        scratch_types=[
            pltpu.VMEM((GSZ, CHUNK), jnp.int32),
            pltpu.VMEM((GSZ, CHUNK), jnp.int32),
            pltpu.VMEM((GSZ, CHUNK), jnp.int32),
            pltpu.VMEM((GSZ, CHUNK), jnp.int32),
            pltpu.VMEM((GSZ, CHUNK), jnp.int32),
            pltpu.VMEM((CHUNK, DH), jnp.float32),
            pltpu.VMEM((CHUNK, DH), jnp.float32),
            pltpu.VMEM_SHARED((NPR, DH), jnp.float32),
            pltpu.SemaphoreType.DMA,
            pltpu.SemaphoreType.DMA,
            pltpu.SemaphoreType.DMA,
            pltpu.SemaphoreType.DMA,
            pltpu.SemaphoreType.DMA,
            pltpu.SemaphoreType.DMA,
        ],
    )
    deg_fn = pl.kernel(
        _sc_deg_body,
        out_type=jax.ShapeDtypeStruct((NC, N, DH), jnp.float32),
        mesh=mesh,
        scratch_types=[
            pltpu.VMEM((GSZD, CHUNK), jnp.int32),
            pltpu.VMEM((GSZD, CHUNK), jnp.int32),
            pltpu.VMEM((CHUNK, DH), jnp.float32),
            pltpu.VMEM_SHARED((NPR, DH), jnp.float32),
            pltpu.SemaphoreType.DMA,
            pltpu.SemaphoreType.DMA,
            pltpu.SemaphoreType.DMA,
            pltpu.SemaphoreType.DMA,
        ],
    )
    return agg_fn, deg_fn


def _sc_segment_sum(x, edge_index):
    agg_fn, deg_fn = _sc_fns()
    x2 = x.reshape(N * NC, DH)
    npt = (EPAD - E) // NS
    pi = jnp.arange(NS * npt, dtype=jnp.int32)
    pad_src = (pi % N).reshape(NS, npt)
    pad_dst = (N + pi % PAD).reshape(NS, npt)
    src_p = jnp.concatenate([edge_index[0].reshape(NS, E // NS), pad_src],
                            axis=1).reshape(-1)
    dst_p = jnp.concatenate([edge_index[1].reshape(NS, E // NS), pad_dst],
                            axis=1).reshape(-1)
    ei2 = jnp.concatenate([src_p, dst_p]).reshape(2 * NCHUNKP, CHUNK)
    zrow = jnp.zeros((ZB, DH), jnp.float32)
    ones = jnp.ones((CHUNK, DH), jnp.float32)
    agg0, agg1 = agg_fn(x2, ei2, zrow)
    deg = deg_fn(ei2, zrow, ones)
    return agg0, agg1, deg


NB = 1000
GRID = N // NB


def _tc1a_body(agg0_ref, agg1_ref, w_ref, p_ref):
    agg = jnp.concatenate([agg0_ref[...], agg1_ref[...]], axis=1)
    p_ref[...] = jnp.dot(agg.astype(jnp.bfloat16),
                         w_ref[...].astype(jnp.bfloat16),
                         preferred_element_type=jnp.float32)


def _tc1_body(p_ref, dp0_ref, dp1_ref, x_ref, b_ref,
              hpre_ref, s1_ref, q1_ref):
    i = pl.program_id(0)
    deg = dp0_ref[0][:, 0:1] + dp1_ref[0][:, 0:1]
    deg = jnp.maximum(deg, 1.0)
    h = jnp.maximum(p_ref[...] / deg + b_ref[...], 0.0) + x_ref[...]
    hpre_ref[...] = h

    @pl.when(i == 0)
    def _():
        s1_ref[...] = jnp.zeros_like(s1_ref)
        q1_ref[...] = jnp.zeros_like(q1_ref)

    s1_ref[...] += jnp.sum(h, axis=0, keepdims=True)
    q1_ref[...] += jnp.sum(h * h, axis=0, keepdims=True)


def _tc2_body(hpre_ref, s1_ref, q1_ref, g1_ref, be1_ref,
              w1_ref, b1_ref, w2_ref, b2_ref, h2_ref, s2_ref, q2_ref):
    i = pl.program_id(0)
    m = s1_ref[...] / N
    var = q1_ref[...] / N - m * m
    rstd = lax.rsqrt(var + 1e-5)
    h = (hpre_ref[...] - m) * rstd * g1_ref[...] + be1_ref[...]
    t = jnp.maximum(
        jnp.dot(h.astype(jnp.bfloat16), w1_ref[...].astype(jnp.bfloat16),
                preferred_element_type=jnp.float32)
        + b1_ref[...], 0.0)
    h2 = (jnp.dot(t.astype(jnp.bfloat16), w2_ref[...].astype(jnp.bfloat16),
                  preferred_element_type=jnp.float32)
          + b2_ref[...]) + h
    h2_ref[...] = h2

    @pl.when(i == 0)
    def _():
        s2_ref[...] = jnp.zeros_like(s2_ref)
        q2_ref[...] = jnp.zeros_like(q2_ref)

    s2_ref[...] += jnp.sum(h2, axis=0, keepdims=True)
    q2_ref[...] += jnp.sum(h2 * h2, axis=0, keepdims=True)


def _tc3_body(h2_ref, s2_ref, q2_ref, g2_ref, be2_ref, out_ref):
    m = s2_ref[...] / N
    var = q2_ref[...] / N - m * m
    rstd = lax.rsqrt(var + 1e-5)
    out_ref[...] = (h2_ref[...] - m) * rstd * g2_ref[...] + be2_ref[...]


def _row_spec(nb, d):
    return pl.BlockSpec((nb, d), lambda i: (i, 0))


def _full_spec(r, c):
    return pl.BlockSpec((r, c), lambda i: (0, 0))


def kernel(x, edge_index, W_gnn, b_gnn, gamma1, beta1, W1, b1, W2, b2,
           gamma2, beta2):
    agg0, agg1, deg = _sc_segment_sum(x, edge_index)

    p = pl.pallas_call(
        _tc1a_body,
        grid=(GRID,),
        in_specs=[
            _row_spec(NB, DH),
            _row_spec(NB, DH),
            _full_spec(D, D),
        ],
        out_specs=_row_spec(NB, D),
        out_shape=jax.ShapeDtypeStruct((N, D), jnp.float32),
    )(agg0, agg1, W_gnn)

    hpre, s1, q1 = pl.pallas_call(
        _tc1_body,
        grid=(GRID,),
        in_specs=[
            _row_spec(NB, D),
            pl.BlockSpec((1, NB, DH), lambda i: (0, i, 0)),
            pl.BlockSpec((1, NB, DH), lambda i: (1, i, 0)),
            _row_spec(NB, D),
            _full_spec(1, D),
        ],
        out_specs=[
            _row_spec(NB, D),
            _full_spec(1, D),
            _full_spec(1, D),
        ],
        out_shape=[
            jax.ShapeDtypeStruct((N, D), jnp.float32),
            jax.ShapeDtypeStruct((1, D), jnp.float32),
            jax.ShapeDtypeStruct((1, D), jnp.float32),
        ],
    )(p, deg, deg, x, b_gnn.reshape(1, D))

    h2, s2, q2 = pl.pallas_call(
        _tc2_body,
        grid=(GRID,),
        in_specs=[
            _row_spec(NB, D),
            _full_spec(1, D),
            _full_spec(1, D),
            _full_spec(1, D),
            _full_spec(1, D),
            _full_spec(D, D),
            _full_spec(1, D),
            _full_spec(D, D),
            _full_spec(1, D),
        ],
        out_specs=[
            _row_spec(NB, D),
            _full_spec(1, D),
            _full_spec(1, D),
        ],
        out_shape=[
            jax.ShapeDtypeStruct((N, D), jnp.float32),
            jax.ShapeDtypeStruct((1, D), jnp.float32),
            jax.ShapeDtypeStruct((1, D), jnp.float32),
        ],
    )(hpre, s1, q1, gamma1.reshape(1, D), beta1.reshape(1, D),
      W1, b1.reshape(1, D), W2, b2.reshape(1, D))

    out = pl.pallas_call(
        _tc3_body,
        grid=(GRID,),
        in_specs=[
            _row_spec(NB, D),
            _full_spec(1, D),
            _full_spec(1, D),
            _full_spec(1, D),
            _full_spec(1, D),
        ],
        out_specs=_row_spec(NB, D),
        out_shape=jax.ShapeDtypeStruct((N, D), jnp.float32),
    )(h2, s2, q2, gamma2.reshape(1, D), beta2.reshape(1, D))

    return out


# speedup vs baseline: 1.0116x; 1.0116x over previous
import functools

import jax
import jax.numpy as jnp
from jax import lax
from jax.experimental import pallas as pl
from jax.experimental.pallas import tpu as pltpu
from jax.experimental.pallas import tpu_sc as plsc

N = 10000
E = 160000
D = 256

NC = 2
NS = 16
DH = D // NC
CHUNK = 128
GSZ = 2
GSZD = 8
NCHUNK_T = 80
NCHUNKP = NS * NCHUNK_T
EPAD = NCHUNKP * CHUNK
NIT = NCHUNK_T // (2 * GSZ)
NIT_DEG = (NCHUNK_T // 2 - GSZD) // (2 * GSZD)
PAD = 48
NPR = N + PAD
ZB = 8
WB = 40


def _sc_agg_body(x2_hbm, ei_hbm, zrow_hbm,
                 agg0_out, agg1_out,
                 src4_a, dst4_a, src4_b, dst4_b, gidx4,
                 r0, r1, agg_sh,
                 isem_a, isem_b, gsem0, gsem1,
                 ssem0, ssem1):
    cid = lax.axis_index("c")
    sid = lax.axis_index("s")
    rows = (r0, r1)
    gsems = (gsem0, gsem1)
    ssems = (ssem0, ssem1)
    base = sid * NCHUNK_T

    @pl.loop(0, (NPR // ZB + NS - 1) // NS)
    def _(it):
        z = it * NS + sid

        @pl.when(z < NPR // ZB)
        def _():
            pltpu.sync_copy(zrow_hbm, agg_sh.at[pl.ds(z * ZB, ZB)])

    plsc.subcore_barrier()

    def fetch_idx(off, src4, dst4, isem):
        pltpu.async_copy(ei_hbm.at[pl.ds(off, GSZ)], src4, isem)
        pltpu.async_copy(ei_hbm.at[pl.ds(NCHUNKP + off, GSZ)], dst4, isem)

    def wait_idx(src4, dst4, isem):
        pltpu.make_async_copy(ei_hbm.at[pl.ds(0, GSZ)], src4, isem).wait()
        pltpu.make_async_copy(ei_hbm.at[pl.ds(0, GSZ)], dst4, isem).wait()

    def compute_gidx(src4):
        for j in range(GSZ):
            @pl.loop(0, CHUNK, step=16)
            def _(i):
                s = src4[j, pl.ds(i, 16)]
                gidx4[j, pl.ds(i, 16)] = s * 2 + cid

    def half_group(dst4, prev_dst4, first):
        for j in range(GSZ):
            if first is None:
                pltpu.make_async_copy(
                    rows[j], agg_sh.at[prev_dst4.at[j]], ssems[j]).wait()
            else:
                @pl.when(first > 0)
                def _():
                    pltpu.make_async_copy(
                        rows[j], agg_sh.at[prev_dst4.at[j]],
                        ssems[j]).wait()

            pltpu.async_copy(x2_hbm.at[gidx4.at[j]], rows[j], gsems[j])

    def scatters(dst4):
        for j in range(GSZ):
            pltpu.make_async_copy(x2_hbm.at[gidx4.at[j]], rows[j],
                                  gsems[j]).wait()
            pltpu.async_copy(rows[j], agg_sh.at[dst4.at[j]], ssems[j],
                             add=True)

    fetch_idx(base, src4_a, dst4_a, isem_a)

    @pl.loop(0, NIT)
    def _(t):
        off_a = base + t * (2 * GSZ)
        wait_idx(src4_a, dst4_a, isem_a)
        compute_gidx(src4_a)
        half_group(dst4_a, dst4_b, t)
        fetch_idx(off_a + GSZ, src4_b, dst4_b, isem_b)
        scatters(dst4_a)
        wait_idx(src4_b, dst4_b, isem_b)
        compute_gidx(src4_b)
        half_group(dst4_b, dst4_a, None)

        @pl.when(t + 1 < NIT)
        def _():
            fetch_idx(off_a + 2 * GSZ, src4_a, dst4_a, isem_a)

        scatters(dst4_b)

    for j in range(GSZ):
        pltpu.make_async_copy(rows[j], agg_sh.at[dst4_b.at[j]],
                              ssems[j]).wait()

    plsc.subcore_barrier()

    @pl.loop(0, (N // WB + NS - 1) // NS)
    def _(it):
        r = it * NS + sid

        @pl.when(r < N // WB)
        def _():
            row = r * WB

            @pl.when(cid == 0)
            def _():
                pltpu.sync_copy(agg_sh.at[pl.ds(row, WB)],
                                agg0_out.at[pl.ds(row, WB)])

            @pl.when(cid == 1)
            def _():
                pltpu.sync_copy(agg_sh.at[pl.ds(row, WB)],
                                agg1_out.at[pl.ds(row, WB)])


def _sc_deg_body(ei_hbm, zrow_hbm, ones_hbm, deg_out,
                 dst4_a, dst4_b, ones_v, deg_sh,
                 isem_a, isem_b, ssem0, ssem1):
    cid = lax.axis_index("c")
    sid = lax.axis_index("s")
    ssems = tuple([ssem0] * (GSZD // 2) + [ssem1] * (GSZD - GSZD // 2))
    base = NCHUNKP + cid * (NCHUNKP // 2) + sid * (NCHUNK_T // 2)
    GS = GSZD

    @pl.loop(0, (NPR // ZB + NS - 1) // NS)
    def _(it):
        z = it * NS + sid

        @pl.when(z < NPR // ZB)
        def _():
            pltpu.sync_copy(zrow_hbm, deg_sh.at[pl.ds(z * ZB, ZB)])

    pltpu.sync_copy(ones_hbm, ones_v)
    plsc.subcore_barrier()

    def wait_idx(dst4, isem):
        pltpu.make_async_copy(ei_hbm.at[pl.ds(0, GS)], dst4, isem).wait()

    def scatter_group(dst4, prev_dst4, first):
        for j in range(GS):
            if first is None:
                pltpu.make_async_copy(
                    ones_v, deg_sh.at[prev_dst4.at[j]], ssems[j]).wait()
            else:
                @pl.when(first > 0)
                def _():
                    pltpu.make_async_copy(
                        ones_v, deg_sh.at[prev_dst4.at[j]], ssems[j]).wait()

        for j in range(GS):
            pltpu.async_copy(ones_v, deg_sh.at[dst4.at[j]], ssems[j],
                             add=True)

    pltpu.async_copy(ei_hbm.at[pl.ds(base, GS)], dst4_a, isem_a)

    @pl.loop(0, NIT_DEG)
    def _(t):
        off_a = base + t * (2 * GS)
        wait_idx(dst4_a, isem_a)
        scatter_group(dst4_a, dst4_b, t)
        pltpu.async_copy(ei_hbm.at[pl.ds(off_a + GS, GS)], dst4_b, isem_b)
        wait_idx(dst4_b, isem_b)
        scatter_group(dst4_b, dst4_a, None)
        pltpu.async_copy(ei_hbm.at[pl.ds(off_a + 2 * GS, GS)],
                         dst4_a, isem_a)

    wait_idx(dst4_a, isem_a)
    scatter_group(dst4_a, dst4_b, None)

    for j in range(GS):
        pltpu.make_async_copy(ones_v, deg_sh.at[dst4_a.at[j]],
                              ssems[j]).wait()

    plsc.subcore_barrier()

    @pl.loop(0, (N // WB + NS - 1) // NS)
    def _(it):
        r = it * NS + sid

        @pl.when(r < N // WB)
        def _():
            row = r * WB
            pltpu.sync_copy(deg_sh.at[pl.ds(row, WB)],
                            deg_out.at[cid, pl.ds(row, WB)])


@functools.cache
def _sc_fns():
    mesh = plsc.VectorSubcoreMesh(core_axis_name="c", subcore_axis_name="s")
    agg_fn = pl.kernel(
        _sc_agg_body,
        out_type=[
            jax.ShapeDtypeStruct((N, DH), jnp.float32),
            jax.ShapeDtypeStruct((N, DH), jnp.float32),
        ],
        mesh=mesh,
        scratch_types=[
            pltpu.VMEM((GSZ, CHUNK), jnp.int32),
            pltpu.VMEM((GSZ, CHUNK), jnp.int32),
            pltpu.VMEM((GSZ, CHUNK), jnp.int32),
            pltpu.VMEM((GSZ, CHUNK), jnp.int32),
            pltpu.VMEM((GSZ, CHUNK), jnp.int32),
            pltpu.VMEM((CHUNK, DH), jnp.float32),
            pltpu.VMEM((CHUNK, DH), jnp.float32),
            pltpu.VMEM_SHARED((NPR, DH), jnp.float32),
            pltpu.SemaphoreType.DMA,
            pltpu.SemaphoreType.DMA,
            pltpu.SemaphoreType.DMA,
            pltpu.SemaphoreType.DMA,
            pltpu.SemaphoreType.DMA,
            pltpu.SemaphoreType.DMA,
        ],
    )
    deg_fn = pl.kernel(
        _sc_deg_body,
        out_type=jax.ShapeDtypeStruct((NC, N, DH), jnp.float32),
        mesh=mesh,
        scratch_types=[
            pltpu.VMEM((GSZD, CHUNK), jnp.int32),
            pltpu.VMEM((GSZD, CHUNK), jnp.int32),
            pltpu.VMEM((CHUNK, DH), jnp.float32),
            pltpu.VMEM_SHARED((NPR, DH), jnp.float32),
            pltpu.SemaphoreType.DMA,
            pltpu.SemaphoreType.DMA,
            pltpu.SemaphoreType.DMA,
            pltpu.SemaphoreType.DMA,
        ],
    )
    return agg_fn, deg_fn


def _sc_segment_sum(x, edge_index):
    agg_fn, deg_fn = _sc_fns()
    x2 = x.reshape(N * NC, DH)
    npt = (EPAD - E) // NS
    pi = jnp.arange(NS * npt, dtype=jnp.int32)
    pad_src = (pi % N).reshape(NS, npt)
    pad_dst = (N + pi % PAD).reshape(NS, npt)
    src_p = jnp.concatenate([edge_index[0].reshape(NS, E // NS), pad_src],
                            axis=1).reshape(-1)
    dst_p = jnp.concatenate([edge_index[1].reshape(NS, E // NS), pad_dst],
                            axis=1).reshape(-1)
    ei2 = jnp.concatenate([src_p, dst_p]).reshape(2 * NCHUNKP, CHUNK)
    zrow = jnp.zeros((ZB, DH), jnp.float32)
    ones = jnp.ones((CHUNK, DH), jnp.float32)
    agg0, agg1 = agg_fn(x2, ei2, zrow)
    deg = deg_fn(ei2, zrow, ones)
    return agg0, agg1, deg


NB = 1000
GRID = N // NB


def _tc1_body(agg0_ref, agg1_ref, dp0_ref, dp1_ref, x_ref, w_ref, b_ref,
              hpre_ref, s1_ref, q1_ref):
    i = pl.program_id(0)
    deg = dp0_ref[0][:, 0:1] + dp1_ref[0][:, 0:1]
    deg = jnp.maximum(deg, 1.0)
    agg = jnp.concatenate([agg0_ref[...], agg1_ref[...]], axis=1)
    mean = agg / deg
    h = jnp.maximum(
        jnp.dot(mean.astype(jnp.bfloat16), w_ref[...].astype(jnp.bfloat16),
                preferred_element_type=jnp.float32)
        + b_ref[...], 0.0) + x_ref[...]
    hpre_ref[...] = h

    @pl.when(i == 0)
    def _():
        s1_ref[...] = jnp.zeros_like(s1_ref)
        q1_ref[...] = jnp.zeros_like(q1_ref)

    s1_ref[...] += jnp.sum(h, axis=0, keepdims=True)
    q1_ref[...] += jnp.sum(h * h, axis=0, keepdims=True)


def _tc2_body(hpre_ref, s1_ref, q1_ref, g1_ref, be1_ref,
              w1_ref, b1_ref, w2_ref, b2_ref, h2_ref, s2_ref, q2_ref):
    i = pl.program_id(0)
    m = s1_ref[...] / N
    var = q1_ref[...] / N - m * m
    rstd = lax.rsqrt(var + 1e-5)
    h = (hpre_ref[...] - m) * rstd * g1_ref[...] + be1_ref[...]
    t = jnp.maximum(
        jnp.dot(h.astype(jnp.bfloat16), w1_ref[...].astype(jnp.bfloat16),
                preferred_element_type=jnp.float32)
        + b1_ref[...], 0.0)
    h2 = (jnp.dot(t.astype(jnp.bfloat16), w2_ref[...].astype(jnp.bfloat16),
                  preferred_element_type=jnp.float32)
          + b2_ref[...]) + h
    h2_ref[...] = h2

    @pl.when(i == 0)
    def _():
        s2_ref[...] = jnp.zeros_like(s2_ref)
        q2_ref[...] = jnp.zeros_like(q2_ref)

    s2_ref[...] += jnp.sum(h2, axis=0, keepdims=True)
    q2_ref[...] += jnp.sum(h2 * h2, axis=0, keepdims=True)


def _tc3_body(h2_ref, s2_ref, q2_ref, g2_ref, be2_ref, out_ref):
    m = s2_ref[...] / N
    var = q2_ref[...] / N - m * m
    rstd = lax.rsqrt(var + 1e-5)
    out_ref[...] = (h2_ref[...] - m) * rstd * g2_ref[...] + be2_ref[...]


def _row_spec(nb, d):
    return pl.BlockSpec((nb, d), lambda i: (i, 0))


def _full_spec(r, c):
    return pl.BlockSpec((r, c), lambda i: (0, 0))


def kernel(x, edge_index, W_gnn, b_gnn, gamma1, beta1, W1, b1, W2, b2,
           gamma2, beta2):
    agg0, agg1, deg = _sc_segment_sum(x, edge_index)

    hpre, s1, q1 = pl.pallas_call(
        _tc1_body,
        grid=(GRID,),
        in_specs=[
            _row_spec(NB, DH),
            _row_spec(NB, DH),
            pl.BlockSpec((1, NB, DH), lambda i: (0, i, 0)),
            pl.BlockSpec((1, NB, DH), lambda i: (1, i, 0)),
            _row_spec(NB, D),
            _full_spec(D, D),
            _full_spec(1, D),
        ],
        out_specs=[
            _row_spec(NB, D),
            _full_spec(1, D),
            _full_spec(1, D),
        ],
        out_shape=[
            jax.ShapeDtypeStruct((N, D), jnp.float32),
            jax.ShapeDtypeStruct((1, D), jnp.float32),
            jax.ShapeDtypeStruct((1, D), jnp.float32),
        ],
    )(agg0, agg1, deg, deg, x, W_gnn, b_gnn.reshape(1, D))

    h2, s2, q2 = pl.pallas_call(
        _tc2_body,
        grid=(GRID,),
        in_specs=[
            _row_spec(NB, D),
            _full_spec(1, D),
            _full_spec(1, D),
            _full_spec(1, D),
            _full_spec(1, D),
            _full_spec(D, D),
            _full_spec(1, D),
            _full_spec(D, D),
            _full_spec(1, D),
        ],
        out_specs=[
            _row_spec(NB, D),
            _full_spec(1, D),
            _full_spec(1, D),
        ],
        out_shape=[
            jax.ShapeDtypeStruct((N, D), jnp.float32),
            jax.ShapeDtypeStruct((1, D), jnp.float32),
            jax.ShapeDtypeStruct((1, D), jnp.float32),
        ],
    )(hpre, s1, q1, gamma1.reshape(1, D), beta1.reshape(1, D),
      W1, b1.reshape(1, D), W2, b2.reshape(1, D))

    out = pl.pallas_call(
        _tc3_body,
        grid=(GRID,),
        in_specs=[
            _row_spec(NB, D),
            _full_spec(1, D),
            _full_spec(1, D),
            _full_spec(1, D),
            _full_spec(1, D),
        ],
        out_specs=_row_spec(NB, D),
        out_shape=jax.ShapeDtypeStruct((N, D), jnp.float32),
    )(h2, s2, q2, gamma2.reshape(1, D), beta2.reshape(1, D))

    return out


# speedup vs baseline: 1.0127x; 1.0011x over previous
import functools

import jax
import jax.numpy as jnp
from jax import lax
from jax.experimental import pallas as pl
from jax.experimental.pallas import tpu as pltpu
from jax.experimental.pallas import tpu_sc as plsc

N = 10000
E = 160000
D = 256

NC = 2
NS = 16
DH = D // NC
CHUNK = 128
GSZ = 2
GSZD = 8
NCHUNK_T = 80
NCHUNKP = NS * NCHUNK_T
EPAD = NCHUNKP * CHUNK
NIT = NCHUNK_T // (2 * GSZ)
NIT_DEG = (NCHUNK_T // 2 - GSZD) // (2 * GSZD)
PAD = 48
NPR = N + PAD
ZB = 8
WB = 40


def _sc_agg_body(x2_hbm, ei_hbm, zrow_hbm,
                 agg0_out, agg1_out,
                 src4_a, dst4_a, src4_b, dst4_b, gidx4,
                 r0, r1, agg_sh,
                 isem_a, isem_b, gsem0, gsem1,
                 ssem0, ssem1):
    cid = lax.axis_index("c")
    sid = lax.axis_index("s")
    rows = (r0, r1)
    gsems = (gsem0, gsem1)
    ssems = (ssem0, ssem1)
    base = sid * NCHUNK_T

    @pl.loop(0, (NPR // ZB + NS - 1) // NS)
    def _(it):
        z = it * NS + sid

        @pl.when(z < NPR // ZB)
        def _():
            pltpu.sync_copy(zrow_hbm, agg_sh.at[pl.ds(z * ZB, ZB)])

    plsc.subcore_barrier()

    def fetch_idx(off, src4, dst4, isem):
        pltpu.async_copy(ei_hbm.at[pl.ds(off, GSZ)], src4, isem)
        pltpu.async_copy(ei_hbm.at[pl.ds(NCHUNKP + off, GSZ)], dst4, isem)

    def wait_idx(src4, dst4, isem):
        pltpu.make_async_copy(ei_hbm.at[pl.ds(0, GSZ)], src4, isem).wait()
        pltpu.make_async_copy(ei_hbm.at[pl.ds(0, GSZ)], dst4, isem).wait()

    def compute_gidx(src4):
        for j in range(GSZ):
            @pl.loop(0, CHUNK, step=16)
            def _(i):
                s = src4[j, pl.ds(i, 16)]
                gidx4[j, pl.ds(i, 16)] = s * 2 + cid

    def half_group(dst4, prev_dst4, first):
        for j in range(GSZ):
            if first is None:
                pltpu.make_async_copy(
                    rows[j], agg_sh.at[prev_dst4.at[j]], ssems[j]).wait()
            else:
                @pl.when(first > 0)
                def _():
                    pltpu.make_async_copy(
                        rows[j], agg_sh.at[prev_dst4.at[j]],
                        ssems[j]).wait()

            pltpu.async_copy(x2_hbm.at[gidx4.at[j]], rows[j], gsems[j])

    def scatters(dst4):
        for j in range(GSZ):
            pltpu.make_async_copy(x2_hbm.at[gidx4.at[j]], rows[j],
                                  gsems[j]).wait()
            pltpu.async_copy(rows[j], agg_sh.at[dst4.at[j]], ssems[j],
                             add=True)

    fetch_idx(base, src4_a, dst4_a, isem_a)

    @pl.loop(0, NIT)
    def _(t):
        off_a = base + t * (2 * GSZ)
        wait_idx(src4_a, dst4_a, isem_a)
        compute_gidx(src4_a)
        half_group(dst4_a, dst4_b, t)
        fetch_idx(off_a + GSZ, src4_b, dst4_b, isem_b)
        scatters(dst4_a)
        wait_idx(src4_b, dst4_b, isem_b)
        compute_gidx(src4_b)
        half_group(dst4_b, dst4_a, None)

        @pl.when(t + 1 < NIT)
        def _():
            fetch_idx(off_a + 2 * GSZ, src4_a, dst4_a, isem_a)

        scatters(dst4_b)

    for j in range(GSZ):
        pltpu.make_async_copy(rows[j], agg_sh.at[dst4_b.at[j]],
                              ssems[j]).wait()

    plsc.subcore_barrier()

    @pl.loop(0, (N // WB + NS - 1) // NS)
    def _(it):
        r = it * NS + sid

        @pl.when(r < N // WB)
        def _():
            row = r * WB

            @pl.when(cid == 0)
            def _():
                pltpu.sync_copy(agg_sh.at[pl.ds(row, WB)],
                                agg0_out.at[pl.ds(row, WB)])

            @pl.when(cid == 1)
            def _():
                pltpu.sync_copy(agg_sh.at[pl.ds(row, WB)],
                                agg1_out.at[pl.ds(row, WB)])


def _sc_deg_body(ei_hbm, zrow_hbm, ones_hbm, deg_out,
                 dst4_a, dst4_b, ones_v, deg_sh,
                 isem_a, isem_b, ssem0, ssem1):
    cid = lax.axis_index("c")
    sid = lax.axis_index("s")
    ssems = tuple([ssem0] * (GSZD // 2) + [ssem1] * (GSZD - GSZD // 2))
    base = NCHUNKP + cid * (NCHUNKP // 2) + sid * (NCHUNK_T // 2)
    GS = GSZD

    @pl.loop(0, (NPR // ZB + NS - 1) // NS)
    def _(it):
        z = it * NS + sid

        @pl.when(z < NPR // ZB)
        def _():
            pltpu.sync_copy(zrow_hbm, deg_sh.at[pl.ds(z * ZB, ZB)])

    pltpu.sync_copy(ones_hbm, ones_v)
    plsc.subcore_barrier()

    def wait_idx(dst4, isem):
        pltpu.make_async_copy(ei_hbm.at[pl.ds(0, GS)], dst4, isem).wait()

    def scatter_group(dst4, prev_dst4, first):
        for j in range(GS):
            if first is None:
                pltpu.make_async_copy(
                    ones_v, deg_sh.at[prev_dst4.at[j]], ssems[j]).wait()
            else:
                @pl.when(first > 0)
                def _():
                    pltpu.make_async_copy(
                        ones_v, deg_sh.at[prev_dst4.at[j]], ssems[j]).wait()

        for j in range(GS):
            pltpu.async_copy(ones_v, deg_sh.at[dst4.at[j]], ssems[j],
                             add=True)

    pltpu.async_copy(ei_hbm.at[pl.ds(base, GS)], dst4_a, isem_a)

    @pl.loop(0, NIT_DEG)
    def _(t):
        off_a = base + t * (2 * GS)
        wait_idx(dst4_a, isem_a)
        scatter_group(dst4_a, dst4_b, t)
        pltpu.async_copy(ei_hbm.at[pl.ds(off_a + GS, GS)], dst4_b, isem_b)
        wait_idx(dst4_b, isem_b)
        scatter_group(dst4_b, dst4_a, None)
        pltpu.async_copy(ei_hbm.at[pl.ds(off_a + 2 * GS, GS)],
                         dst4_a, isem_a)

    wait_idx(dst4_a, isem_a)
    scatter_group(dst4_a, dst4_b, None)

    for j in range(GS):
        pltpu.make_async_copy(ones_v, deg_sh.at[dst4_a.at[j]],
                              ssems[j]).wait()

    plsc.subcore_barrier()

    @pl.loop(0, (N // WB + NS - 1) // NS)
    def _(it):
        r = it * NS + sid

        @pl.when(r < N // WB)
        def _():
            row = r * WB
            pltpu.sync_copy(deg_sh.at[pl.ds(row, WB)],
                            deg_out.at[cid, pl.ds(row, WB)])


@functools.cache
def _sc_fns():
    mesh = plsc.VectorSubcoreMesh(core_axis_name="c", subcore_axis_name="s")
    agg_fn = pl.kernel(
        _sc_agg_body,
        out_type=[
            jax.ShapeDtypeStruct((N, DH), jnp.float32),
            jax.ShapeDtypeStruct((N, DH), jnp.float32),
        ],
        mesh=mesh,
        scratch_types=[
            pltpu.VMEM((GSZ, CHUNK), jnp.int32),
            pltpu.VMEM((GSZ, CHUNK), jnp.int32),
            pltpu.VMEM((GSZ, CHUNK), jnp.int32),
            pltpu.VMEM((GSZ, CHUNK), jnp.int32),
            pltpu.VMEM((GSZ, CHUNK), jnp.int32),
            pltpu.VMEM((CHUNK, DH), jnp.float32),
            pltpu.VMEM((CHUNK, DH), jnp.float32),
            pltpu.VMEM_SHARED((NPR, DH), jnp.float32),
            pltpu.SemaphoreType.DMA,
            pltpu.SemaphoreType.DMA,
            pltpu.SemaphoreType.DMA,
            pltpu.SemaphoreType.DMA,
            pltpu.SemaphoreType.DMA,
            pltpu.SemaphoreType.DMA,
        ],
    )
    deg_fn = pl.kernel(
        _sc_deg_body,
        out_type=jax.ShapeDtypeStruct((NC, N, DH), jnp.float32),
        mesh=mesh,
        scratch_types=[
            pltpu.VMEM((GSZD, CHUNK), jnp.int32),
            pltpu.VMEM((GSZD, CHUNK), jnp.int32),
            pltpu.VMEM((CHUNK, DH), jnp.float32),
            pltpu.VMEM_SHARED((NPR, DH), jnp.float32),
            pltpu.SemaphoreType.DMA,
            pltpu.SemaphoreType.DMA,
            pltpu.SemaphoreType.DMA,
            pltpu.SemaphoreType.DMA,
        ],
    )
    return agg_fn, deg_fn


def _sc_segment_sum(x, edge_index):
    agg_fn, deg_fn = _sc_fns()
    x2 = x.reshape(N * NC, DH)
    npt = (EPAD - E) // NS
    pi = jnp.arange(NS * npt, dtype=jnp.int32)
    pad_src = (pi % N).reshape(NS, npt)
    pad_dst = (N + pi % PAD).reshape(NS, npt)
    src_p = jnp.concatenate([edge_index[0].reshape(NS, E // NS), pad_src],
                            axis=1).reshape(-1)
    dst_p = jnp.concatenate([edge_index[1].reshape(NS, E // NS), pad_dst],
                            axis=1).reshape(-1)
    ei2 = jnp.concatenate([src_p, dst_p]).reshape(2 * NCHUNKP, CHUNK)
    zrow = jnp.zeros((ZB, DH), jnp.float32)
    ones = jnp.ones((CHUNK, DH), jnp.float32)
    agg0, agg1 = agg_fn(x2, ei2, zrow)
    deg = deg_fn(ei2, zrow, ones)
    return agg0, agg1, deg


NB = 1000
GRID = N // NB


def _tc1_body(agg0_ref, agg1_ref, dp0_ref, dp1_ref, x_ref, w_ref, b_ref,
              hpre_ref, s1_ref, q1_ref):
    i = pl.program_id(0)
    deg = dp0_ref[0][:, 0:1] + dp1_ref[0][:, 0:1]
    deg = jnp.maximum(deg, 1.0)
    agg = jnp.concatenate([agg0_ref[...], agg1_ref[...]], axis=1)
    mean = agg / deg
    h = jnp.maximum(
        jnp.dot(mean, w_ref[...], preferred_element_type=jnp.float32)
        + b_ref[...], 0.0) + x_ref[...]
    hpre_ref[...] = h

    @pl.when(i == 0)
    def _():
        s1_ref[...] = jnp.zeros_like(s1_ref)
        q1_ref[...] = jnp.zeros_like(q1_ref)

    s1_ref[...] += jnp.sum(h, axis=0, keepdims=True)
    q1_ref[...] += jnp.sum(h * h, axis=0, keepdims=True)


def _tc2_body(hpre_ref, s1_ref, q1_ref, g1_ref, be1_ref,
              w1_ref, b1_ref, w2_ref, b2_ref, h2_ref, s2_ref, q2_ref):
    i = pl.program_id(0)
    m = s1_ref[...] / N
    var = q1_ref[...] / N - m * m
    rstd = lax.rsqrt(var + 1e-5)
    h = (hpre_ref[...] - m) * rstd * g1_ref[...] + be1_ref[...]
    t = jnp.maximum(
        jnp.dot(h, w1_ref[...], preferred_element_type=jnp.float32)
        + b1_ref[...], 0.0)
    h2 = (jnp.dot(t, w2_ref[...], preferred_element_type=jnp.float32)
          + b2_ref[...]) + h
    h2_ref[...] = h2

    @pl.when(i == 0)
    def _():
        s2_ref[...] = jnp.zeros_like(s2_ref)
        q2_ref[...] = jnp.zeros_like(q2_ref)

    s2_ref[...] += jnp.sum(h2, axis=0, keepdims=True)
    q2_ref[...] += jnp.sum(h2 * h2, axis=0, keepdims=True)


def _tc3_body(h2_ref, s2_ref, q2_ref, g2_ref, be2_ref, out_ref):
    m = s2_ref[...] / N
    var = q2_ref[...] / N - m * m
    rstd = lax.rsqrt(var + 1e-5)
    out_ref[...] = (h2_ref[...] - m) * rstd * g2_ref[...] + be2_ref[...]


def _row_spec(nb, d):
    return pl.BlockSpec((nb, d), lambda i: (i, 0))


def _full_spec(r, c):
    return pl.BlockSpec((r, c), lambda i: (0, 0))


def kernel(x, edge_index, W_gnn, b_gnn, gamma1, beta1, W1, b1, W2, b2,
           gamma2, beta2):
    agg0, agg1, deg = _sc_segment_sum(x, edge_index)

    hpre, s1, q1 = pl.pallas_call(
        _tc1_body,
        grid=(GRID,),
        in_specs=[
            _row_spec(NB, DH),
            _row_spec(NB, DH),
            pl.BlockSpec((1, NB, DH), lambda i: (0, i, 0)),
            pl.BlockSpec((1, NB, DH), lambda i: (1, i, 0)),
            _row_spec(NB, D),
            _full_spec(D, D),
            _full_spec(1, D),
        ],
        out_specs=[
            _row_spec(NB, D),
            _full_spec(1, D),
            _full_spec(1, D),
        ],
        out_shape=[
            jax.ShapeDtypeStruct((N, D), jnp.float32),
            jax.ShapeDtypeStruct((1, D), jnp.float32),
            jax.ShapeDtypeStruct((1, D), jnp.float32),
        ],
    )(agg0, agg1, deg, deg, x, W_gnn, b_gnn.reshape(1, D))

    h2, s2, q2 = pl.pallas_call(
        _tc2_body,
        grid=(GRID,),
        in_specs=[
            _row_spec(NB, D),
            _full_spec(1, D),
            _full_spec(1, D),
            _full_spec(1, D),
            _full_spec(1, D),
            _full_spec(D, D),
            _full_spec(1, D),
            _full_spec(D, D),
            _full_spec(1, D),
        ],
        out_specs=[
            _row_spec(NB, D),
            _full_spec(1, D),
            _full_spec(1, D),
        ],
        out_shape=[
            jax.ShapeDtypeStruct((N, D), jnp.float32),
            jax.ShapeDtypeStruct((1, D), jnp.float32),
            jax.ShapeDtypeStruct((1, D), jnp.float32),
        ],
    )(hpre, s1, q1, gamma1.reshape(1, D), beta1.reshape(1, D),
      W1, b1.reshape(1, D), W2, b2.reshape(1, D))

    out = pl.pallas_call(
        _tc3_body,
        grid=(GRID,),
        in_specs=[
            _row_spec(NB, D),
            _full_spec(1, D),
            _full_spec(1, D),
            _full_spec(1, D),
            _full_spec(1, D),
        ],
        out_specs=_row_spec(NB, D),
        out_shape=jax.ShapeDtypeStruct((N, D), jnp.float32),
    )(h2, s2, q2, gamma2.reshape(1, D), beta2.reshape(1, D))

    return out


# speedup vs baseline: 1.0215x; 1.0087x over previous
import functools

import jax
import jax.numpy as jnp
from jax import lax
from jax.experimental import pallas as pl
from jax.experimental.pallas import tpu as pltpu
from jax.experimental.pallas import tpu_sc as plsc

N = 10000
E = 160000
D = 256

NC = 2
NS = 16
DH = D // NC
CHUNK = 128
GSZ = 2
GSZD = 8
NCHUNK_T = 80
NCHUNKP = NS * NCHUNK_T
EPAD = NCHUNKP * CHUNK
NIT = NCHUNK_T // (2 * GSZ)
NIT_DEG = (NCHUNK_T // 2 - GSZD) // (2 * GSZD)
PAD = 48
NPR = N + PAD
ZB = 8
WB = 40


def _sc_agg_body(x2_hbm, ei_hbm, zrow_hbm, ones_hbm,
                 agg0_out, agg1_out, deg_out,
                 src4_a, dst4_a, src4_b, dst4_b, gidx4,
                 r0, r1, agg_sh,
                 isem_a, isem_b, gsem0, gsem1,
                 ssem0, ssem1):
    cid = lax.axis_index("c")
    sid = lax.axis_index("s")
    rows = (r0, r1)
    gsems = (gsem0, gsem1)
    ssems = (ssem0, ssem1)
    base = sid * NCHUNK_T

    @pl.loop(0, (NPR // ZB + NS - 1) // NS)
    def _(it):
        z = it * NS + sid

        @pl.when(z < NPR // ZB)
        def _():
            pltpu.sync_copy(zrow_hbm, agg_sh.at[pl.ds(z * ZB, ZB)])

    plsc.subcore_barrier()

    def fetch_idx(off, src4, dst4, isem):
        pltpu.async_copy(ei_hbm.at[pl.ds(off, GSZ)], src4, isem)
        pltpu.async_copy(ei_hbm.at[pl.ds(NCHUNKP + off, GSZ)], dst4, isem)

    def wait_idx(src4, dst4, isem):
        pltpu.make_async_copy(ei_hbm.at[pl.ds(0, GSZ)], src4, isem).wait()
        pltpu.make_async_copy(ei_hbm.at[pl.ds(0, GSZ)], dst4, isem).wait()

    def compute_gidx(src4):
        for j in range(GSZ):
            @pl.loop(0, CHUNK, step=16)
            def _(i):
                s = src4[j, pl.ds(i, 16)]
                gidx4[j, pl.ds(i, 16)] = s * 2 + cid

    def half_group(dst4, prev_dst4, first):
        for j in range(GSZ):
            if first is None:
                pltpu.make_async_copy(
                    rows[j], agg_sh.at[prev_dst4.at[j]], ssems[j]).wait()
            else:
                @pl.when(first > 0)
                def _():
                    pltpu.make_async_copy(
                        rows[j], agg_sh.at[prev_dst4.at[j]],
                        ssems[j]).wait()

            pltpu.async_copy(x2_hbm.at[gidx4.at[j]], rows[j], gsems[j])

    def scatters(dst4):
        for j in range(GSZ):
            pltpu.make_async_copy(x2_hbm.at[gidx4.at[j]], rows[j],
                                  gsems[j]).wait()
            pltpu.async_copy(rows[j], agg_sh.at[dst4.at[j]], ssems[j],
                             add=True)

    fetch_idx(base, src4_a, dst4_a, isem_a)

    @pl.loop(0, NIT)
    def _(t):
        off_a = base + t * (2 * GSZ)
        wait_idx(src4_a, dst4_a, isem_a)
        compute_gidx(src4_a)
        half_group(dst4_a, dst4_b, t)
        fetch_idx(off_a + GSZ, src4_b, dst4_b, isem_b)
        scatters(dst4_a)
        wait_idx(src4_b, dst4_b, isem_b)
        compute_gidx(src4_b)
        half_group(dst4_b, dst4_a, None)

        @pl.when(t + 1 < NIT)
        def _():
            fetch_idx(off_a + 2 * GSZ, src4_a, dst4_a, isem_a)

        scatters(dst4_b)

    for j in range(GSZ):
        pltpu.make_async_copy(rows[j], agg_sh.at[dst4_b.at[j]],
                              ssems[j]).wait()

    plsc.subcore_barrier()

    @pl.loop(0, (N // WB + NS - 1) // NS)
    def _(it):
        r = it * NS + sid

        @pl.when(r < N // WB)
        def _():
            row = r * WB

            @pl.when(cid == 0)
            def _():
                pltpu.sync_copy(agg_sh.at[pl.ds(row, WB)],
                                agg0_out.at[pl.ds(row, WB)])

            @pl.when(cid == 1)
            def _():
                pltpu.sync_copy(agg_sh.at[pl.ds(row, WB)],
                                agg1_out.at[pl.ds(row, WB)])

    plsc.subcore_barrier()

    @pl.loop(0, (NPR // ZB + NS - 1) // NS)
    def _(it):
        z = it * NS + sid

        @pl.when(z < NPR // ZB)
        def _():
            pltpu.sync_copy(zrow_hbm, agg_sh.at[pl.ds(z * ZB, ZB)])

    pltpu.sync_copy(ones_hbm, r0)
    plsc.subcore_barrier()

    dbase = NCHUNKP + cid * (NCHUNKP // 2) + sid * (NCHUNK_T // 2)
    NITD = NCHUNK_T // 2 // (2 * GSZ)

    def wait_didx(dst4, isem):
        pltpu.make_async_copy(ei_hbm.at[pl.ds(0, GSZ)], dst4, isem).wait()

    def deg_scatters(dst4, prev_dst4, first):
        for j in range(GSZ):
            if first is None:
                pltpu.make_async_copy(
                    r0, agg_sh.at[prev_dst4.at[j]], ssems[j]).wait()
            else:
                @pl.when(first > 0)
                def _():
                    pltpu.make_async_copy(
                        r0, agg_sh.at[prev_dst4.at[j]], ssems[j]).wait()

        for j in range(GSZ):
            pltpu.async_copy(r0, agg_sh.at[dst4.at[j]], ssems[j], add=True)

    pltpu.async_copy(ei_hbm.at[pl.ds(dbase, GSZ)], dst4_a, isem_a)

    @pl.loop(0, NITD)
    def _(t):
        off = dbase + t * (2 * GSZ)
        wait_didx(dst4_a, isem_a)
        deg_scatters(dst4_a, dst4_b, t)
        pltpu.async_copy(ei_hbm.at[pl.ds(off + GSZ, GSZ)], dst4_b, isem_b)
        wait_didx(dst4_b, isem_b)
        deg_scatters(dst4_b, dst4_a, None)

        @pl.when(t + 1 < NITD)
        def _():
            pltpu.async_copy(ei_hbm.at[pl.ds(off + 2 * GSZ, GSZ)],
                             dst4_a, isem_a)

    for j in range(GSZ):
        pltpu.make_async_copy(r0, agg_sh.at[dst4_b.at[j]], ssems[j]).wait()

    plsc.subcore_barrier()

    @pl.loop(0, (N // WB + NS - 1) // NS)
    def _(it):
        r = it * NS + sid

        @pl.when(r < N // WB)
        def _():
            row = r * WB
            pltpu.sync_copy(agg_sh.at[pl.ds(row, WB)],
                            deg_out.at[cid, pl.ds(row, WB)])


@functools.cache
def _sc_fns():
    mesh = plsc.VectorSubcoreMesh(core_axis_name="c", subcore_axis_name="s")
    agg_fn = pl.kernel(
        _sc_agg_body,
        out_type=[
            jax.ShapeDtypeStruct((N, DH), jnp.float32),
            jax.ShapeDtypeStruct((N, DH), jnp.float32),
            jax.ShapeDtypeStruct((NC, N, DH), jnp.float32),
        ],
        mesh=mesh,
        scratch_types=[
            pltpu.VMEM((GSZ, CHUNK), jnp.int32),
            pltpu.VMEM((GSZ, CHUNK), jnp.int32),
            pltpu.VMEM((GSZ, CHUNK), jnp.int32),
            pltpu.VMEM((GSZ, CHUNK), jnp.int32),
            pltpu.VMEM((GSZ, CHUNK), jnp.int32),
            pltpu.VMEM((CHUNK, DH), jnp.float32),
            pltpu.VMEM((CHUNK, DH), jnp.float32),
            pltpu.VMEM_SHARED((NPR, DH), jnp.float32),
            pltpu.SemaphoreType.DMA,
            pltpu.SemaphoreType.DMA,
            pltpu.SemaphoreType.DMA,
            pltpu.SemaphoreType.DMA,
            pltpu.SemaphoreType.DMA,
            pltpu.SemaphoreType.DMA,
        ],
    )
    return agg_fn


def _sc_segment_sum(x, edge_index):
    agg_fn = _sc_fns()
    x2 = x.reshape(N * NC, DH)
    npt = (EPAD - E) // NS
    pi = jnp.arange(NS * npt, dtype=jnp.int32)
    pad_src = (pi % N).reshape(NS, npt)
    pad_dst = (N + pi % PAD).reshape(NS, npt)
    src_p = jnp.concatenate([edge_index[0].reshape(NS, E // NS), pad_src],
                            axis=1).reshape(-1)
    dst_p = jnp.concatenate([edge_index[1].reshape(NS, E // NS), pad_dst],
                            axis=1).reshape(-1)
    ei2 = jnp.concatenate([src_p, dst_p]).reshape(2 * NCHUNKP, CHUNK)
    zrow = jnp.zeros((ZB, DH), jnp.float32)
    ones = jnp.ones((CHUNK, DH), jnp.float32)
    agg0, agg1, deg = agg_fn(x2, ei2, zrow, ones)
    return agg0, agg1, deg


NB = 1000
GRID = N // NB


def _tc1_body(agg0_ref, agg1_ref, dp0_ref, dp1_ref, x_ref, w_ref, b_ref,
              hpre_ref, s1_ref, q1_ref):
    i = pl.program_id(0)
    deg = dp0_ref[0][:, 0:1] + dp1_ref[0][:, 0:1]
    deg = jnp.maximum(deg, 1.0)
    agg = jnp.concatenate([agg0_ref[...], agg1_ref[...]], axis=1)
    mean = agg / deg
    h = jnp.maximum(
        jnp.dot(mean, w_ref[...], preferred_element_type=jnp.float32)
        + b_ref[...], 0.0) + x_ref[...]
    hpre_ref[...] = h

    @pl.when(i == 0)
    def _():
        s1_ref[...] = jnp.zeros_like(s1_ref)
        q1_ref[...] = jnp.zeros_like(q1_ref)

    s1_ref[...] += jnp.sum(h, axis=0, keepdims=True)
    q1_ref[...] += jnp.sum(h * h, axis=0, keepdims=True)


def _tc2_body(hpre_ref, s1_ref, q1_ref, g1_ref, be1_ref,
              w1_ref, b1_ref, w2_ref, b2_ref, h2_ref, s2_ref, q2_ref):
    i = pl.program_id(0)
    m = s1_ref[...] / N
    var = q1_ref[...] / N - m * m
    rstd = lax.rsqrt(var + 1e-5)
    h = (hpre_ref[...] - m) * rstd * g1_ref[...] + be1_ref[...]
    t = jnp.maximum(
        jnp.dot(h, w1_ref[...], preferred_element_type=jnp.float32)
        + b1_ref[...], 0.0)
    h2 = (jnp.dot(t, w2_ref[...], preferred_element_type=jnp.float32)
          + b2_ref[...]) + h
    h2_ref[...] = h2

    @pl.when(i == 0)
    def _():
        s2_ref[...] = jnp.zeros_like(s2_ref)
        q2_ref[...] = jnp.zeros_like(q2_ref)

    s2_ref[...] += jnp.sum(h2, axis=0, keepdims=True)
    q2_ref[...] += jnp.sum(h2 * h2, axis=0, keepdims=True)


def _tc3_body(h2_ref, s2_ref, q2_ref, g2_ref, be2_ref, out_ref):
    m = s2_ref[...] / N
    var = q2_ref[...] / N - m * m
    rstd = lax.rsqrt(var + 1e-5)
    out_ref[...] = (h2_ref[...] - m) * rstd * g2_ref[...] + be2_ref[...]


def _row_spec(nb, d):
    return pl.BlockSpec((nb, d), lambda i: (i, 0))


def _full_spec(r, c):
    return pl.BlockSpec((r, c), lambda i: (0, 0))


def kernel(x, edge_index, W_gnn, b_gnn, gamma1, beta1, W1, b1, W2, b2,
           gamma2, beta2):
    agg0, agg1, deg = _sc_segment_sum(x, edge_index)

    hpre, s1, q1 = pl.pallas_call(
        _tc1_body,
        grid=(GRID,),
        in_specs=[
            _row_spec(NB, DH),
            _row_spec(NB, DH),
            pl.BlockSpec((1, NB, DH), lambda i: (0, i, 0)),
            pl.BlockSpec((1, NB, DH), lambda i: (1, i, 0)),
            _row_spec(NB, D),
            _full_spec(D, D),
            _full_spec(1, D),
        ],
        out_specs=[
            _row_spec(NB, D),
            _full_spec(1, D),
            _full_spec(1, D),
        ],
        out_shape=[
            jax.ShapeDtypeStruct((N, D), jnp.float32),
            jax.ShapeDtypeStruct((1, D), jnp.float32),
            jax.ShapeDtypeStruct((1, D), jnp.float32),
        ],
    )(agg0, agg1, deg, deg, x, W_gnn, b_gnn.reshape(1, D))

    h2, s2, q2 = pl.pallas_call(
        _tc2_body,
        grid=(GRID,),
        in_specs=[
            _row_spec(NB, D),
            _full_spec(1, D),
            _full_spec(1, D),
            _full_spec(1, D),
            _full_spec(1, D),
            _full_spec(D, D),
            _full_spec(1, D),
            _full_spec(D, D),
            _full_spec(1, D),
        ],
        out_specs=[
            _row_spec(NB, D),
            _full_spec(1, D),
            _full_spec(1, D),
        ],
        out_shape=[
            jax.ShapeDtypeStruct((N, D), jnp.float32),
            jax.ShapeDtypeStruct((1, D), jnp.float32),
            jax.ShapeDtypeStruct((1, D), jnp.float32),
        ],
    )(hpre, s1, q1, gamma1.reshape(1, D), beta1.reshape(1, D),
      W1, b1.reshape(1, D), W2, b2.reshape(1, D))

    out = pl.pallas_call(
        _tc3_body,
        grid=(GRID,),
        in_specs=[
            _row_spec(NB, D),
            _full_spec(1, D),
            _full_spec(1, D),
            _full_spec(1, D),
            _full_spec(1, D),
        ],
        out_specs=_row_spec(NB, D),
        out_shape=jax.ShapeDtypeStruct((N, D), jnp.float32),
    )(h2, s2, q2, gamma2.reshape(1, D), beta2.reshape(1, D))

    return out
